# Initial kernel scaffold; baseline (speedup 1.0000x reference)
import functools

import jax
import jax.numpy as jnp
from jax import lax
from jax.experimental import pallas as pl
from jax.experimental.pallas import tpu as pltpu
from jax.experimental.pallas import tpu_sc as plsc

N = 10000
E = 320000
D = 128
L = 16
NC, NS = 2, 16
NW = NC * NS
EPW = E // NW
C = 80
NCHUNK = EPW // C
RPT = N // NS

_mesh = plsc.VectorSubcoreMesh(
    core_axis_name="c", subcore_axis_name="s", num_cores=NC, num_subcores=NS
)


@functools.partial(
    pl.kernel,
    out_type=jax.ShapeDtypeStruct((NC, N, D), jnp.float32),
    mesh=_mesh,
    scratch_types=[
        pltpu.VMEM((NCHUNK, C), jnp.int32),
        pltpu.VMEM((NCHUNK, C), jnp.int32),
        pltpu.VMEM((NCHUNK, C), jnp.float32),
        pltpu.VMEM((C, D), jnp.float32),
        pltpu.VMEM((125, D), jnp.float32),
        pltpu.VMEM_SHARED((N, D), jnp.float32),
        pltpu.SemaphoreType.DMA,
    ],
)
def _spmm_sc(src_hbm, dst_hbm, val_hbm, x_hbm, out_hbm,
             src_v, dst_v, val_v, rows_v, zero_v, acc_sh, sem):
    cid = lax.axis_index("c")
    sid = lax.axis_index("s")
    wid = sid * NC + cid

    pltpu.sync_copy(src_hbm.at[wid], src_v)
    pltpu.sync_copy(dst_hbm.at[wid], dst_v)
    pltpu.sync_copy(val_hbm.at[wid], val_v)

    zvec = jnp.zeros((L,), jnp.float32)

    def _zrow(i, carry):
        for k in range(D // L):
            zero_v[i, pl.ds(k * L, L)] = zvec
        return carry

    lax.fori_loop(0, 125, _zrow, 0)
    for t in range(RPT // 125):
        pltpu.sync_copy(zero_v, acc_sh.at[pl.ds(sid * RPT + t * 125, 125)])
    plsc.subcore_barrier()

    def _chunk(j, carry):
        pltpu.async_copy(x_hbm.at[src_v.at[j]], rows_v, sem).wait()

        def _edge(e, c2):
            v = val_v[j, e]
            for k in range(D // L):
                sl = pl.ds(k * L, L)
                rows_v[e, sl] = rows_v[e, sl] * v
            return c2

        lax.fori_loop(0, C, _edge, 0)
        pltpu.sync_copy(rows_v, acc_sh.at[dst_v.at[j]], add=True)
        return carry

    lax.fori_loop(0, NCHUNK, _chunk, 0)
    plsc.subcore_barrier()

    pltpu.sync_copy(acc_sh.at[pl.ds(sid * RPT, RPT)],
                    out_hbm.at[cid, pl.ds(sid * RPT, RPT)])


_BR = 2500


def _inter_body(p_ref, f_ref, o_ref):
    o_ref[...] = (p_ref[0] + p_ref[1]) * f_ref[...]


_inter = pl.pallas_call(
    _inter_body,
    grid=(N // _BR,),
    in_specs=[
        pl.BlockSpec((NC, _BR, D), lambda i: (0, i, 0)),
        pl.BlockSpec((_BR, D), lambda i: (i, 0)),
    ],
    out_specs=pl.BlockSpec((_BR, D), lambda i: (i, 0)),
    out_shape=jax.ShapeDtypeStruct((N, D), jnp.float32),
)


def _final_body(p_ref, q_ref, f_ref, w1_ref, w2_ref, b_ref, o_ref):
    x1 = p_ref[0] + p_ref[1] + f_ref[...]
    x2 = q_ref[0] + q_ref[1]
    y = lax.dot_general(x1, w1_ref[...], (((1,), (1,)), ((), ())),
                        preferred_element_type=jnp.float32)
    y = y + lax.dot_general(x2, w2_ref[...], (((1,), (1,)), ((), ())),
                            preferred_element_type=jnp.float32)
    o_ref[...] = y + b_ref[...]


_final = pl.pallas_call(
    _final_body,
    grid=(N // _BR,),
    in_specs=[
        pl.BlockSpec((NC, _BR, D), lambda i: (0, i, 0)),
        pl.BlockSpec((NC, _BR, D), lambda i: (0, i, 0)),
        pl.BlockSpec((_BR, D), lambda i: (i, 0)),
        pl.BlockSpec((D, D), lambda i: (0, 0)),
        pl.BlockSpec((D, D), lambda i: (0, 0)),
        pl.BlockSpec((1, D), lambda i: (0, 0)),
    ],
    out_specs=pl.BlockSpec((_BR, D), lambda i: (i, 0)),
    out_shape=jax.ShapeDtypeStruct((N, D), jnp.float32),
)


def kernel(laplacian_indices, laplacian_values, features, W1, b1, W2, b2):
    dst = laplacian_indices[0].reshape(NW, NCHUNK, C)
    src = laplacian_indices[1].reshape(NW, NCHUNK, C)
    vals = laplacian_values.reshape(NW, NCHUNK, C)
    p = _spmm_sc(src, dst, vals, features)
    inter = _inter(p, features)
    q = _spmm_sc(src, dst, vals, inter)
    return _final(p, q, features, W1, W2, (b1 + b2).reshape(1, D))


# speedup vs baseline: 3.0726x; 3.0726x over previous
import functools

import jax
import jax.numpy as jnp
from jax import lax
from jax.experimental import pallas as pl
from jax.experimental.pallas import tpu as pltpu
from jax.experimental.pallas import tpu_sc as plsc

N = 10000
E = 320000
D = 128
L = 16
NC, NS = 2, 16
NW = NC * NS
C = 128
NCHUNK = 80
EPW = NCHUNK * C
EPAD = NW * EPW
SLAB = 640
SLAB_LAST = N - (NS - 1) * SLAB

_mesh = plsc.VectorSubcoreMesh(
    core_axis_name="c", subcore_axis_name="s", num_cores=NC, num_subcores=NS
)


@functools.partial(
    pl.kernel,
    out_type=jax.ShapeDtypeStruct((NC, N, D), jnp.float32),
    mesh=_mesh,
    scratch_types=[
        pltpu.VMEM((NCHUNK, C), jnp.int32),
        pltpu.VMEM((NCHUNK, C), jnp.float32),
        pltpu.VMEM((C,), jnp.int32),
        pltpu.VMEM((C,), jnp.int32),
        pltpu.VMEM((C, D), jnp.float32),
        pltpu.VMEM_SHARED((N, D), jnp.float32),
        pltpu.SemaphoreType.DMA,
    ],
)
def _spmm_sc(pk_hbm, val_hbm, x_hbm, out_hbm,
             pk_v, val_v, src_c, dst_c, rows_v, acc_sh, sem):
    cid = lax.axis_index("c")
    sid = lax.axis_index("s")
    wid = sid * NC + cid

    pltpu.sync_copy(pk_hbm.at[wid], pk_v)
    pltpu.sync_copy(val_hbm.at[wid], val_v)

    zvec = jnp.zeros((L,), jnp.float32)

    def _zrow(i, carry):
        for k in range(D // L):
            rows_v[i, pl.ds(k * L, L)] = zvec
        return carry

    lax.fori_loop(0, C, _zrow, 0)

    @pl.when(sid < NS - 1)
    def _zero_main():
        for t in range(SLAB // C):
            pltpu.sync_copy(rows_v, acc_sh.at[pl.ds(sid * SLAB + t * C, C)])

    @pl.when(sid == NS - 1)
    def _zero_last():
        base = (NS - 1) * SLAB
        for t in range(SLAB_LAST // C):
            pltpu.sync_copy(rows_v, acc_sh.at[pl.ds(base + t * C, C)])
        rem = SLAB_LAST % C
        pltpu.sync_copy(rows_v.at[pl.ds(0, rem)],
                        acc_sh.at[pl.ds(base + (SLAB_LAST // C) * C, rem)])

    plsc.subcore_barrier()

    mask14 = jnp.full((L,), (1 << 14) - 1, jnp.int32)
    sh14 = jnp.full((L,), 14, jnp.int32)

    def _chunk(j, carry):
        for g in range(C // L):
            sl = pl.ds(g * L, L)
            pk = pk_v[j, sl]
            src_c[sl] = pk & mask14
            dst_c[sl] = lax.shift_right_logical(pk, sh14)
        pltpu.async_copy(x_hbm.at[src_c], rows_v, sem).wait()

        def _group(g, c2):
            vv = val_v[j, pl.ds(g * L, L)]
            for i in range(L):
                e = g * L + i
                v = vv[i]
                for k in range(D // L):
                    sl = pl.ds(k * L, L)
                    rows_v[e, sl] = rows_v[e, sl] * v
            return c2

        lax.fori_loop(0, C // L, _group, 0)
        pltpu.sync_copy(rows_v, acc_sh.at[dst_c], add=True)
        return carry

    lax.fori_loop(0, NCHUNK, _chunk, 0)
    plsc.subcore_barrier()

    @pl.when(sid < NS - 1)
    def _pub():
        pltpu.sync_copy(acc_sh.at[pl.ds(sid * SLAB, SLAB)],
                        out_hbm.at[cid, pl.ds(sid * SLAB, SLAB)])

    @pl.when(sid == NS - 1)
    def _pub_last():
        pltpu.sync_copy(acc_sh.at[pl.ds((NS - 1) * SLAB, SLAB_LAST)],
                        out_hbm.at[cid, pl.ds((NS - 1) * SLAB, SLAB_LAST)])


_BR = 2000


def _inter_body(p_ref, f_ref, o_ref):
    o_ref[...] = (p_ref[0] + p_ref[1]) * f_ref[...]


_inter = pl.pallas_call(
    _inter_body,
    grid=(N // _BR,),
    in_specs=[
        pl.BlockSpec((NC, _BR, D), lambda i: (0, i, 0)),
        pl.BlockSpec((_BR, D), lambda i: (i, 0)),
    ],
    out_specs=pl.BlockSpec((_BR, D), lambda i: (i, 0)),
    out_shape=jax.ShapeDtypeStruct((N, D), jnp.float32),
)


def _final_body(p_ref, q_ref, f_ref, w1_ref, w2_ref, b_ref, o_ref):
    x1 = p_ref[0] + p_ref[1] + f_ref[...]
    x2 = q_ref[0] + q_ref[1]
    y = lax.dot_general(x1, w1_ref[...], (((1,), (1,)), ((), ())),
                        preferred_element_type=jnp.float32)
    y = y + lax.dot_general(x2, w2_ref[...], (((1,), (1,)), ((), ())),
                            preferred_element_type=jnp.float32)
    o_ref[...] = y + b_ref[...]


_final = pl.pallas_call(
    _final_body,
    grid=(N // _BR,),
    in_specs=[
        pl.BlockSpec((NC, _BR, D), lambda i: (0, i, 0)),
        pl.BlockSpec((NC, _BR, D), lambda i: (0, i, 0)),
        pl.BlockSpec((_BR, D), lambda i: (i, 0)),
        pl.BlockSpec((D, D), lambda i: (0, 0)),
        pl.BlockSpec((D, D), lambda i: (0, 0)),
        pl.BlockSpec((1, D), lambda i: (0, 0)),
    ],
    out_specs=pl.BlockSpec((_BR, D), lambda i: (i, 0)),
    out_shape=jax.ShapeDtypeStruct((N, D), jnp.float32),
)


def kernel(laplacian_indices, laplacian_values, features, W1, b1, W2, b2):
    packed = (laplacian_indices[0] << 14) | laplacian_indices[1]
    packed = jnp.concatenate(
        [packed, jnp.zeros((EPAD - E,), jnp.int32)]).reshape(NW, NCHUNK, C)
    vals = jnp.concatenate(
        [laplacian_values,
         jnp.zeros((EPAD - E,), jnp.float32)]).reshape(NW, NCHUNK, C)
    p = _spmm_sc(packed, vals, features)
    inter = _inter(p, features)
    q = _spmm_sc(packed, vals, inter)
    return _final(p, q, features, W1, W2, (b1 + b2).reshape(1, D))


# speedup vs baseline: 3.4974x; 1.1382x over previous
import functools

import jax
import jax.numpy as jnp
from jax import lax
from jax.experimental import pallas as pl
from jax.experimental.pallas import tpu as pltpu
from jax.experimental.pallas import tpu_sc as plsc

N = 10000
E = 320000
D = 128
L = 16
NC, NS = 2, 16
NW = NC * NS
C = 128
NCHUNK = 80
NPAIR = NCHUNK // 2
EPW = NCHUNK * C
EPAD = NW * EPW
SLAB = 640
SLAB_LAST = N - (NS - 1) * SLAB

_mesh = plsc.VectorSubcoreMesh(
    core_axis_name="c", subcore_axis_name="s", num_cores=NC, num_subcores=NS
)


@functools.partial(
    pl.kernel,
    out_type=jax.ShapeDtypeStruct((NC, N, D), jnp.float32),
    mesh=_mesh,
    scratch_types=[
        pltpu.VMEM((NCHUNK, C), jnp.int32),
        pltpu.VMEM((C, D), jnp.float32),
        pltpu.VMEM((C, D), jnp.float32),
        pltpu.VMEM((C,), jnp.int32),
        pltpu.VMEM((C,), jnp.int32),
        pltpu.VMEM((C,), jnp.int32),
        pltpu.VMEM((C,), jnp.int32),
        pltpu.VMEM((C,), jnp.float32),
        pltpu.VMEM((C,), jnp.float32),
        pltpu.VMEM_SHARED((N, D), jnp.float32),
        pltpu.SemaphoreType.DMA,
        pltpu.SemaphoreType.DMA,
        pltpu.SemaphoreType.DMA,
        pltpu.SemaphoreType.DMA,
        pltpu.SemaphoreType.DMA,
        pltpu.SemaphoreType.DMA,
        pltpu.SemaphoreType.DMA,
    ],
)
def _spmm_sc(pk_hbm, val_hbm, x_hbm, out_hbm,
             pk_v, rows0, rows1, src0, src1, dst0, dst1, val0, val1,
             acc_sh, g0, g1, s0, s1, v0, v1, zs):
    cid = lax.axis_index("c")
    sid = lax.axis_index("s")
    wid = sid * NC + cid

    rows = (rows0, rows1)
    src = (src0, src1)
    dst = (dst0, dst1)
    val = (val0, val1)
    gsem = (g0, g1)
    ssem = (s0, s1)
    vsem = (v0, v1)

    pltpu.sync_copy(pk_hbm.at[wid], pk_v)

    zvec = jnp.zeros((L,), jnp.float32)

    def _zrow(i, carry):
        for k in range(D // L):
            rows0[i, pl.ds(k * L, L)] = zvec
        return carry

    lax.fori_loop(0, C, _zrow, 0)

    @pl.when(sid < NS - 1)
    def _zero_main():
        for t in range(SLAB // C):
            pltpu.async_copy(
                rows0, acc_sh.at[pl.ds(sid * SLAB + t * C, C)], zs)
        for t in range(SLAB // C):
            pltpu.make_async_copy(
                rows0, acc_sh.at[pl.ds(sid * SLAB + t * C, C)], zs).wait()

    @pl.when(sid == NS - 1)
    def _zero_last():
        base = (NS - 1) * SLAB
        nfull = SLAB_LAST // C
        rem = SLAB_LAST % C
        for t in range(nfull):
            pltpu.async_copy(rows0, acc_sh.at[pl.ds(base + t * C, C)], zs)
        pltpu.async_copy(rows0.at[pl.ds(0, rem)],
                         acc_sh.at[pl.ds(base + nfull * C, rem)], zs)
        for t in range(nfull):
            pltpu.make_async_copy(
                rows0, acc_sh.at[pl.ds(base + t * C, C)], zs).wait()
        pltpu.make_async_copy(rows0.at[pl.ds(0, rem)],
                              acc_sh.at[pl.ds(base + nfull * C, rem)],
                              zs).wait()

    plsc.subcore_barrier()

    mask14 = jnp.full((L,), (1 << 14) - 1, jnp.int32)
    sh14 = jnp.full((L,), 14, jnp.int32)

    def _unpack(j, b):
        for g in range(C // L):
            sl = pl.ds(g * L, L)
            pk = pk_v[j, sl]
            src[b][sl] = pk & mask14
            dst[b][sl] = lax.shift_right_logical(pk, sh14)

    def _launch(j, b):
        _unpack(j, b)
        pltpu.async_copy(x_hbm.at[src[b]], rows[b], gsem[b])
        pltpu.async_copy(val_hbm.at[wid, j], val[b], vsem[b])

    def _scale(b):
        def _group(g, c2):
            vv = val[b][pl.ds(g * L, L)]
            for i in range(L):
                e = g * L + i
                v = vv[i]
                for k in range(D // L):
                    sl = pl.ds(k * L, L)
                    rows[b][e, sl] = rows[b][e, sl] * v
            return c2

        lax.fori_loop(0, C // L, _group, 0)

    def _process(b, j):
        pltpu.make_async_copy(x_hbm.at[src[b]], rows[b], gsem[b]).wait()
        pltpu.make_async_copy(val_hbm.at[wid, j], val[b], vsem[b]).wait()
        _scale(b)
        pltpu.async_copy(rows[b], acc_sh.at[dst[b]], ssem[b], add=True)

    def _drain(b):
        pltpu.make_async_copy(rows[b], acc_sh.at[dst[b]], ssem[b]).wait()

    _launch(0, 0)
    _launch(1, 1)

    def _pair(m, carry):
        j0 = 2 * m
        _process(0, j0)
        _process(1, j0 + 1)
        _drain(0)

        @pl.when(j0 + 2 < NCHUNK)
        def _next0():
            _launch(j0 + 2, 0)

        _drain(1)

        @pl.when(j0 + 3 < NCHUNK)
        def _next1():
            _launch(j0 + 3, 1)

        return carry

    lax.fori_loop(0, NPAIR, _pair, 0)
    plsc.subcore_barrier()

    @pl.when(sid < NS - 1)
    def _pub():
        pltpu.sync_copy(acc_sh.at[pl.ds(sid * SLAB, SLAB)],
                        out_hbm.at[cid, pl.ds(sid * SLAB, SLAB)])

    @pl.when(sid == NS - 1)
    def _pub_last():
        pltpu.sync_copy(acc_sh.at[pl.ds((NS - 1) * SLAB, SLAB_LAST)],
                        out_hbm.at[cid, pl.ds((NS - 1) * SLAB, SLAB_LAST)])


_BR = 2000


def _inter_body(p_ref, f_ref, o_ref):
    o_ref[...] = (p_ref[0] + p_ref[1]) * f_ref[...]


_inter = pl.pallas_call(
    _inter_body,
    grid=(N // _BR,),
    in_specs=[
        pl.BlockSpec((NC, _BR, D), lambda i: (0, i, 0)),
        pl.BlockSpec((_BR, D), lambda i: (i, 0)),
    ],
    out_specs=pl.BlockSpec((_BR, D), lambda i: (i, 0)),
    out_shape=jax.ShapeDtypeStruct((N, D), jnp.float32),
)


def _part1_body(p_ref, f_ref, w1_ref, b_ref, o_ref):
    x1 = p_ref[0] + p_ref[1] + f_ref[...]
    y = lax.dot_general(x1, w1_ref[...], (((1,), (1,)), ((), ())),
                        preferred_element_type=jnp.float32)
    o_ref[...] = y + b_ref[...]


_part1 = pl.pallas_call(
    _part1_body,
    grid=(N // _BR,),
    in_specs=[
        pl.BlockSpec((NC, _BR, D), lambda i: (0, i, 0)),
        pl.BlockSpec((_BR, D), lambda i: (i, 0)),
        pl.BlockSpec((D, D), lambda i: (0, 0)),
        pl.BlockSpec((1, D), lambda i: (0, 0)),
    ],
    out_specs=pl.BlockSpec((_BR, D), lambda i: (i, 0)),
    out_shape=jax.ShapeDtypeStruct((N, D), jnp.float32),
)


def _part2_body(a_ref, q_ref, w2_ref, o_ref):
    x2 = q_ref[0] + q_ref[1]
    y = lax.dot_general(x2, w2_ref[...], (((1,), (1,)), ((), ())),
                        preferred_element_type=jnp.float32)
    o_ref[...] = a_ref[...] + y


_part2 = pl.pallas_call(
    _part2_body,
    grid=(N // _BR,),
    in_specs=[
        pl.BlockSpec((_BR, D), lambda i: (i, 0)),
        pl.BlockSpec((NC, _BR, D), lambda i: (0, i, 0)),
        pl.BlockSpec((D, D), lambda i: (0, 0)),
    ],
    out_specs=pl.BlockSpec((_BR, D), lambda i: (i, 0)),
    out_shape=jax.ShapeDtypeStruct((N, D), jnp.float32),
)


def kernel(laplacian_indices, laplacian_values, features, W1, b1, W2, b2):
    packed = (laplacian_indices[0] << 14) | laplacian_indices[1]
    packed = jnp.concatenate(
        [packed, jnp.zeros((EPAD - E,), jnp.int32)]).reshape(NW, NCHUNK, C)
    vals = jnp.concatenate(
        [laplacian_values,
         jnp.zeros((EPAD - E,), jnp.float32)]).reshape(NW, NCHUNK, C)
    p = _spmm_sc(packed, vals, features)
    inter = _inter(p, features)
    parta = _part1(p, features, W1, (b1 + b2).reshape(1, D))
    q = _spmm_sc(packed, vals, inter)
    return _part2(parta, q, W2)


# speedup vs baseline: 3.4994x; 1.0006x over previous
import functools

import jax
import jax.numpy as jnp
from jax import lax
from jax.experimental import pallas as pl
from jax.experimental.pallas import tpu as pltpu
from jax.experimental.pallas import tpu_sc as plsc

N = 10000
E = 320000
D = 128
L = 16
NC, NS = 2, 16
NW = NC * NS
C = 128
NCHUNK = 80
NPAIR = NCHUNK // 2
EPW = NCHUNK * C
EPAD = NW * EPW
SLAB = 640
SLAB_LAST = N - (NS - 1) * SLAB

_mesh = plsc.VectorSubcoreMesh(
    core_axis_name="c", subcore_axis_name="s", num_cores=NC, num_subcores=NS
)


@functools.partial(
    pl.kernel,
    out_type=jax.ShapeDtypeStruct((NC, N, D), jnp.float32),
    mesh=_mesh,
    scratch_types=[
        pltpu.VMEM((NCHUNK, C), jnp.int32),
        pltpu.VMEM((C, D), jnp.float32),
        pltpu.VMEM((C, D), jnp.float32),
        pltpu.VMEM((C,), jnp.int32),
        pltpu.VMEM((C,), jnp.int32),
        pltpu.VMEM((C,), jnp.int32),
        pltpu.VMEM((C,), jnp.int32),
        pltpu.VMEM((C,), jnp.float32),
        pltpu.VMEM((C,), jnp.float32),
        pltpu.VMEM_SHARED((N, D), jnp.float32),
        pltpu.SemaphoreType.DMA,
        pltpu.SemaphoreType.DMA,
        pltpu.SemaphoreType.DMA,
        pltpu.SemaphoreType.DMA,
        pltpu.SemaphoreType.DMA,
        pltpu.SemaphoreType.DMA,
        pltpu.SemaphoreType.DMA,
    ],
)
def _spmm_sc(pk_hbm, val_hbm, x_hbm, out_hbm,
             pk_v, rows0, rows1, src0, src1, dst0, dst1, val0, val1,
             acc_sh, g0, g1, s0, s1, v0, v1, zs):
    cid = lax.axis_index("c")
    sid = lax.axis_index("s")
    wid = sid * NC + cid

    rows = (rows0, rows1)
    src = (src0, src1)
    dst = (dst0, dst1)
    val = (val0, val1)
    gsem = (g0, g1)
    ssem = (s0, s1)
    vsem = (v0, v1)

    with jax.named_scope("stage_pk"):
        pltpu.sync_copy(pk_hbm.at[wid], pk_v)

    zvec = jnp.zeros((L,), jnp.float32)

    def _zrow(i, carry):
        for k in range(D // L):
            rows0[i, pl.ds(k * L, L)] = zvec
        return carry

    lax.fori_loop(0, C, _zrow, 0)

    @pl.when(sid < NS - 1)
    def _zero_main():
        for t in range(SLAB // C):
            pltpu.async_copy(
                rows0, acc_sh.at[pl.ds(sid * SLAB + t * C, C)], zs)
        for t in range(SLAB // C):
            pltpu.make_async_copy(
                rows0, acc_sh.at[pl.ds(sid * SLAB + t * C, C)], zs).wait()

    @pl.when(sid == NS - 1)
    def _zero_last():
        base = (NS - 1) * SLAB
        nfull = SLAB_LAST // C
        rem = SLAB_LAST % C
        for t in range(nfull):
            pltpu.async_copy(rows0, acc_sh.at[pl.ds(base + t * C, C)], zs)
        pltpu.async_copy(rows0.at[pl.ds(0, rem)],
                         acc_sh.at[pl.ds(base + nfull * C, rem)], zs)
        for t in range(nfull):
            pltpu.make_async_copy(
                rows0, acc_sh.at[pl.ds(base + t * C, C)], zs).wait()
        pltpu.make_async_copy(rows0.at[pl.ds(0, rem)],
                              acc_sh.at[pl.ds(base + nfull * C, rem)],
                              zs).wait()

    with jax.named_scope("zero_barrier"):
        plsc.subcore_barrier()

    mask14 = jnp.full((L,), (1 << 14) - 1, jnp.int32)
    sh14 = jnp.full((L,), 14, jnp.int32)

    def _unpack(j, b):
        for g in range(C // L):
            sl = pl.ds(g * L, L)
            pk = pk_v[j, sl]
            src[b][sl] = pk & mask14
            dst[b][sl] = lax.shift_right_logical(pk, sh14)

    def _launch(j, b):
        _unpack(j, b)
        pltpu.async_copy(x_hbm.at[src[b]], rows[b], gsem[b])
        pltpu.async_copy(val_hbm.at[wid, j], val[b], vsem[b])

    def _scale(b):
        def _group(g, c2):
            vv = val[b][pl.ds(g * L, L)]
            for i in range(L):
                e = g * L + i
                v = vv[i]
                for k in range(D // L):
                    sl = pl.ds(k * L, L)
                    rows[b][e, sl] = rows[b][e, sl] * v
            return c2

        lax.fori_loop(0, C // L, _group, 0)

    def _process(b, j):
        pltpu.make_async_copy(x_hbm.at[src[b]], rows[b], gsem[b]).wait()
        pltpu.make_async_copy(val_hbm.at[wid, j], val[b], vsem[b]).wait()
        _scale(b)
        pltpu.async_copy(rows[b], acc_sh.at[dst[b]], ssem[b], add=True)

    def _drain(b):
        pltpu.make_async_copy(rows[b], acc_sh.at[dst[b]], ssem[b]).wait()

    _launch(0, 0)
    _launch(1, 1)

    def _pair(m, carry):
        j0 = 2 * m
        _process(0, j0)
        _process(1, j0 + 1)
        _drain(0)

        @pl.when(j0 + 2 < NCHUNK)
        def _next0():
            _launch(j0 + 2, 0)

        _drain(1)

        @pl.when(j0 + 3 < NCHUNK)
        def _next1():
            _launch(j0 + 3, 1)

        return carry

    with jax.named_scope("edge_loop"):
        lax.fori_loop(0, NPAIR, _pair, 0)
    with jax.named_scope("end_barrier"):
        plsc.subcore_barrier()

    @pl.when(sid < NS - 1)
    def _pub():
        with jax.named_scope("publish"):
            pltpu.sync_copy(acc_sh.at[pl.ds(sid * SLAB, SLAB)],
                            out_hbm.at[cid, pl.ds(sid * SLAB, SLAB)])

    @pl.when(sid == NS - 1)
    def _pub_last():
        pltpu.sync_copy(acc_sh.at[pl.ds((NS - 1) * SLAB, SLAB_LAST)],
                        out_hbm.at[cid, pl.ds((NS - 1) * SLAB, SLAB_LAST)])


_BR = 2000


def _inter_body(p_ref, f_ref, o_ref):
    o_ref[...] = (p_ref[0] + p_ref[1]) * f_ref[...]


_inter = pl.pallas_call(
    _inter_body,
    grid=(N // _BR,),
    in_specs=[
        pl.BlockSpec((NC, _BR, D), lambda i: (0, i, 0)),
        pl.BlockSpec((_BR, D), lambda i: (i, 0)),
    ],
    out_specs=pl.BlockSpec((_BR, D), lambda i: (i, 0)),
    out_shape=jax.ShapeDtypeStruct((N, D), jnp.float32),
)


def _part1_body(p_ref, f_ref, w1_ref, b_ref, o_ref):
    x1 = p_ref[0] + p_ref[1] + f_ref[...]
    y = lax.dot_general(x1, w1_ref[...], (((1,), (1,)), ((), ())),
                        preferred_element_type=jnp.float32)
    o_ref[...] = y + b_ref[...]


_part1 = pl.pallas_call(
    _part1_body,
    grid=(N // _BR,),
    in_specs=[
        pl.BlockSpec((NC, _BR, D), lambda i: (0, i, 0)),
        pl.BlockSpec((_BR, D), lambda i: (i, 0)),
        pl.BlockSpec((D, D), lambda i: (0, 0)),
        pl.BlockSpec((1, D), lambda i: (0, 0)),
    ],
    out_specs=pl.BlockSpec((_BR, D), lambda i: (i, 0)),
    out_shape=jax.ShapeDtypeStruct((N, D), jnp.float32),
)


def _part2_body(a_ref, q_ref, w2_ref, o_ref):
    x2 = q_ref[0] + q_ref[1]
    y = lax.dot_general(x2, w2_ref[...], (((1,), (1,)), ((), ())),
                        preferred_element_type=jnp.float32)
    o_ref[...] = a_ref[...] + y


_part2 = pl.pallas_call(
    _part2_body,
    grid=(N // _BR,),
    in_specs=[
        pl.BlockSpec((_BR, D), lambda i: (i, 0)),
        pl.BlockSpec((NC, _BR, D), lambda i: (0, i, 0)),
        pl.BlockSpec((D, D), lambda i: (0, 0)),
    ],
    out_specs=pl.BlockSpec((_BR, D), lambda i: (i, 0)),
    out_shape=jax.ShapeDtypeStruct((N, D), jnp.float32),
)


def kernel(laplacian_indices, laplacian_values, features, W1, b1, W2, b2):
    packed = (laplacian_indices[0] << 14) | laplacian_indices[1]
    packed = jnp.concatenate(
        [packed, jnp.zeros((EPAD - E,), jnp.int32)]).reshape(NW, NCHUNK, C)
    vals = jnp.concatenate(
        [laplacian_values,
         jnp.zeros((EPAD - E,), jnp.float32)]).reshape(NW, NCHUNK, C)
    p = _spmm_sc(packed, vals, features)
    inter = _inter(p, features)
    parta = _part1(p, features, W1, (b1 + b2).reshape(1, D))
    q = _spmm_sc(packed, vals, inter)
    return _part2(parta, q, W2)


# speedup vs baseline: 10.4760x; 2.9937x over previous
import functools

import jax
import jax.numpy as jnp
from jax import lax
from jax.experimental import pallas as pl
from jax.experimental.pallas import tpu as pltpu
from jax.experimental.pallas import tpu_sc as plsc

N = 10000
E = 320000
D = 128
L = 16
NC, NS = 2, 16
NW = NC * NS
EPW = E // NW
C = 128
NCHUNK = EPW // C
NPAIR = NCHUNK // 2
TAIL = NCHUNK * C
TAILN = EPW - TAIL
SLAB = 640
SLAB_LAST = N - (NS - 1) * SLAB

_mesh = plsc.VectorSubcoreMesh(
    core_axis_name="c", subcore_axis_name="s", num_cores=NC, num_subcores=NS
)


@functools.partial(
    pl.kernel,
    out_type=jax.ShapeDtypeStruct((NC, N, D), jnp.float32),
    mesh=_mesh,
    scratch_types=[
        pltpu.VMEM((EPW,), jnp.int32),
        pltpu.VMEM((C, D), jnp.float32),
        pltpu.VMEM((C, D), jnp.float32),
        pltpu.VMEM((C,), jnp.int32),
        pltpu.VMEM((C,), jnp.int32),
        pltpu.VMEM((C,), jnp.float32),
        pltpu.VMEM((C,), jnp.float32),
        pltpu.VMEM((TAILN,), jnp.int32),
        pltpu.VMEM((TAILN,), jnp.float32),
        pltpu.VMEM_SHARED((N, D), jnp.float32),
        pltpu.SemaphoreType.DMA,
        pltpu.SemaphoreType.DMA,
        pltpu.SemaphoreType.DMA,
        pltpu.SemaphoreType.DMA,
        pltpu.SemaphoreType.DMA,
        pltpu.SemaphoreType.DMA,
        pltpu.SemaphoreType.DMA,
        pltpu.SemaphoreType.DMA,
        pltpu.SemaphoreType.DMA,
    ],
)
def _spmm_sc(dst_hbm, src_hbm, val_hbm, x_hbm, out_hbm,
             src_v, rows0, rows1, dst0, dst1, val0, val1, dstt, valt,
             acc_sh, g0, g1, s0, s1, d0, d1, v0, v1, zs):
    cid = lax.axis_index("c")
    sid = lax.axis_index("s")
    wid = sid * NC + cid
    base = wid * EPW

    rows = (rows0, rows1)
    dst = (dst0, dst1)
    val = (val0, val1)
    gsem = (g0, g1)
    ssem = (s0, s1)
    dsem = (d0, d1)
    vsem = (v0, v1)

    with jax.named_scope("stage_src"):
        pltpu.sync_copy(src_hbm.at[pl.ds(base, EPW)], src_v)

    zvec = jnp.zeros((L,), jnp.float32)

    def _zrow(i, carry):
        for k in range(D // L):
            rows0[i, pl.ds(k * L, L)] = zvec
        return carry

    lax.fori_loop(0, C, _zrow, 0)

    @pl.when(sid < NS - 1)
    def _zero_main():
        for t in range(SLAB // C):
            pltpu.async_copy(
                rows0, acc_sh.at[pl.ds(sid * SLAB + t * C, C)], zs)
        for t in range(SLAB // C):
            pltpu.make_async_copy(
                rows0, acc_sh.at[pl.ds(sid * SLAB + t * C, C)], zs).wait()

    @pl.when(sid == NS - 1)
    def _zero_last():
        zbase = (NS - 1) * SLAB
        nfull = SLAB_LAST // C
        rem = SLAB_LAST % C
        for t in range(nfull):
            pltpu.async_copy(rows0, acc_sh.at[pl.ds(zbase + t * C, C)], zs)
        pltpu.async_copy(rows0.at[pl.ds(0, rem)],
                         acc_sh.at[pl.ds(zbase + nfull * C, rem)], zs)
        for t in range(nfull):
            pltpu.make_async_copy(
                rows0, acc_sh.at[pl.ds(zbase + t * C, C)], zs).wait()
        pltpu.make_async_copy(rows0.at[pl.ds(0, rem)],
                              acc_sh.at[pl.ds(zbase + nfull * C, rem)],
                              zs).wait()

    with jax.named_scope("zero_barrier"):
        plsc.subcore_barrier()

    def _launch(j, b):
        pltpu.async_copy(x_hbm.at[src_v.at[pl.ds(j * C, C)]], rows[b],
                         gsem[b])
        pltpu.async_copy(dst_hbm.at[pl.ds(base + j * C, C)], dst[b], dsem[b])
        pltpu.async_copy(val_hbm.at[pl.ds(base + j * C, C)], val[b], vsem[b])

    def _scale(b):
        def _group(g, c2):
            vv = val[b][pl.ds(g * L, L)]
            for i in range(L):
                e = g * L + i
                v = vv[i]
                for k in range(D // L):
                    sl = pl.ds(k * L, L)
                    rows[b][e, sl] = rows[b][e, sl] * v
            return c2

        lax.fori_loop(0, C // L, _group, 0)

    def _process(b, j):
        pltpu.make_async_copy(x_hbm.at[src_v.at[pl.ds(j * C, C)]], rows[b],
                              gsem[b]).wait()
        pltpu.make_async_copy(dst_hbm.at[pl.ds(base + j * C, C)], dst[b],
                              dsem[b]).wait()
        pltpu.make_async_copy(val_hbm.at[pl.ds(base + j * C, C)], val[b],
                              vsem[b]).wait()
        _scale(b)
        pltpu.async_copy(rows[b], acc_sh.at[dst[b]], ssem[b], add=True)

    def _drain(b):
        pltpu.make_async_copy(rows[b], acc_sh.at[dst[b]], ssem[b]).wait()

    _launch(0, 0)
    _launch(1, 1)

    def _pair(m, carry):
        j0 = 2 * m
        _process(0, j0)
        _process(1, j0 + 1)
        _drain(0)

        @pl.when(j0 + 2 < NCHUNK)
        def _next0():
            _launch(j0 + 2, 0)

        _drain(1)

        @pl.when(j0 + 3 < NCHUNK)
        def _next1():
            _launch(j0 + 3, 1)

        return carry

    with jax.named_scope("edge_loop"):
        lax.fori_loop(0, NPAIR, _pair, 0)

    with jax.named_scope("tail"):
        pltpu.sync_copy(dst_hbm.at[pl.ds(base + TAIL, TAILN)], dstt)
        pltpu.sync_copy(val_hbm.at[pl.ds(base + TAIL, TAILN)], valt)
        pltpu.async_copy(x_hbm.at[src_v.at[pl.ds(TAIL, TAILN)]],
                         rows0.at[pl.ds(0, TAILN)], g0).wait()
        vv = valt[...]
        for i in range(TAILN):
            v = vv[i]
            for k in range(D // L):
                sl = pl.ds(k * L, L)
                rows0[i, sl] = rows0[i, sl] * v
        pltpu.sync_copy(rows0.at[pl.ds(0, TAILN)], acc_sh.at[dstt], add=True)

    with jax.named_scope("end_barrier"):
        plsc.subcore_barrier()

    @pl.when(sid < NS - 1)
    def _pub():
        with jax.named_scope("publish"):
            pltpu.sync_copy(acc_sh.at[pl.ds(sid * SLAB, SLAB)],
                            out_hbm.at[cid, pl.ds(sid * SLAB, SLAB)])

    @pl.when(sid == NS - 1)
    def _pub_last():
        pltpu.sync_copy(acc_sh.at[pl.ds((NS - 1) * SLAB, SLAB_LAST)],
                        out_hbm.at[cid, pl.ds((NS - 1) * SLAB, SLAB_LAST)])


_BR = 2000


def _inter_body(p_ref, f_ref, o_ref):
    o_ref[...] = (p_ref[0] + p_ref[1]) * f_ref[...]


_inter = pl.pallas_call(
    _inter_body,
    grid=(N // _BR,),
    in_specs=[
        pl.BlockSpec((NC, _BR, D), lambda i: (0, i, 0)),
        pl.BlockSpec((_BR, D), lambda i: (i, 0)),
    ],
    out_specs=pl.BlockSpec((_BR, D), lambda i: (i, 0)),
    out_shape=jax.ShapeDtypeStruct((N, D), jnp.float32),
)


def _part1_body(p_ref, f_ref, w1_ref, b_ref, o_ref):
    x1 = p_ref[0] + p_ref[1] + f_ref[...]
    y = lax.dot_general(x1, w1_ref[...], (((1,), (1,)), ((), ())),
                        preferred_element_type=jnp.float32)
    o_ref[...] = y + b_ref[...]


_part1 = pl.pallas_call(
    _part1_body,
    grid=(N // _BR,),
    in_specs=[
        pl.BlockSpec((NC, _BR, D), lambda i: (0, i, 0)),
        pl.BlockSpec((_BR, D), lambda i: (i, 0)),
        pl.BlockSpec((D, D), lambda i: (0, 0)),
        pl.BlockSpec((1, D), lambda i: (0, 0)),
    ],
    out_specs=pl.BlockSpec((_BR, D), lambda i: (i, 0)),
    out_shape=jax.ShapeDtypeStruct((N, D), jnp.float32),
)


def _part2_body(a_ref, q_ref, w2_ref, o_ref):
    x2 = q_ref[0] + q_ref[1]
    y = lax.dot_general(x2, w2_ref[...], (((1,), (1,)), ((), ())),
                        preferred_element_type=jnp.float32)
    o_ref[...] = a_ref[...] + y


_part2 = pl.pallas_call(
    _part2_body,
    grid=(N // _BR,),
    in_specs=[
        pl.BlockSpec((_BR, D), lambda i: (i, 0)),
        pl.BlockSpec((NC, _BR, D), lambda i: (0, i, 0)),
        pl.BlockSpec((D, D), lambda i: (0, 0)),
    ],
    out_specs=pl.BlockSpec((_BR, D), lambda i: (i, 0)),
    out_shape=jax.ShapeDtypeStruct((N, D), jnp.float32),
)


def kernel(laplacian_indices, laplacian_values, features, W1, b1, W2, b2):
    dst_arr = laplacian_indices[0]
    src_arr = laplacian_indices[1]
    p = _spmm_sc(dst_arr, src_arr, laplacian_values, features)
    inter = _inter(p, features)
    parta = _part1(p, features, W1, (b1 + b2).reshape(1, D))
    q = _spmm_sc(dst_arr, src_arr, laplacian_values, inter)
    return _part2(parta, q, W2)


# speedup vs baseline: 10.4832x; 1.0007x over previous
import functools

import jax
import jax.numpy as jnp
from jax import lax
from jax.experimental import pallas as pl
from jax.experimental.pallas import tpu as pltpu
from jax.experimental.pallas import tpu_sc as plsc

N = 10000
E = 320000
D = 128
L = 16
NC, NS = 2, 16
NW = NC * NS
EPW = E // NW
C = 128
NCHUNK = EPW // C
NPAIR = NCHUNK // 2
TAIL = NCHUNK * C
TAILN = EPW - TAIL
SLAB = 640
SLAB_LAST = N - (NS - 1) * SLAB

_mesh = plsc.VectorSubcoreMesh(
    core_axis_name="c", subcore_axis_name="s", num_cores=NC, num_subcores=NS
)


@functools.partial(
    pl.kernel,
    out_type=jax.ShapeDtypeStruct((NC, N, D), jnp.float32),
    mesh=_mesh,
    scratch_types=[
        pltpu.VMEM((EPW,), jnp.int32),
        pltpu.VMEM((C, D), jnp.float32),
        pltpu.VMEM((C, D), jnp.float32),
        pltpu.VMEM((C // 2,), jnp.int32),
        pltpu.VMEM((C // 2,), jnp.int32),
        pltpu.VMEM((C // 2,), jnp.int32),
        pltpu.VMEM((C // 2,), jnp.int32),
        pltpu.VMEM((C,), jnp.float32),
        pltpu.VMEM((C,), jnp.float32),
        pltpu.VMEM((TAILN,), jnp.int32),
        pltpu.VMEM((TAILN,), jnp.float32),
        pltpu.VMEM_SHARED((N, D), jnp.float32),
        pltpu.SemaphoreType.DMA,
        pltpu.SemaphoreType.DMA,
        pltpu.SemaphoreType.DMA,
        pltpu.SemaphoreType.DMA,
        pltpu.SemaphoreType.DMA,
        pltpu.SemaphoreType.DMA,
        pltpu.SemaphoreType.DMA,
        pltpu.SemaphoreType.DMA,
        pltpu.SemaphoreType.DMA,
    ],
)
def _spmm_sc(dst_hbm, src_hbm, val_hbm, x_hbm, out_hbm,
             src_v, rows0, rows1, da0, db0, da1, db1, val0, val1, dstt, valt,
             acc_sh, g0, g1, s0, s1, d0, d1, v0, v1, zs):
    cid = lax.axis_index("c")
    sid = lax.axis_index("s")
    wid = sid * NC + cid
    base = wid * EPW

    rows = (rows0, rows1)
    dsta = (da0, da1)
    dstb = (db0, db1)
    val = (val0, val1)
    gsem = (g0, g1)
    ssem = (s0, s1)
    dsem = (d0, d1)
    vsem = (v0, v1)

    with jax.named_scope("stage_src"):
        pltpu.sync_copy(src_hbm.at[pl.ds(base, EPW)], src_v)

    zvec = jnp.zeros((L,), jnp.float32)

    def _zrow(i, carry):
        for k in range(D // L):
            rows0[i, pl.ds(k * L, L)] = zvec
        return carry

    lax.fori_loop(0, C, _zrow, 0)

    @pl.when(sid < NS - 1)
    def _zero_main():
        for t in range(SLAB // C):
            pltpu.async_copy(
                rows0, acc_sh.at[pl.ds(sid * SLAB + t * C, C)], zs)
        for t in range(SLAB // C):
            pltpu.make_async_copy(
                rows0, acc_sh.at[pl.ds(sid * SLAB + t * C, C)], zs).wait()

    @pl.when(sid == NS - 1)
    def _zero_last():
        zbase = (NS - 1) * SLAB
        nfull = SLAB_LAST // C
        rem = SLAB_LAST % C
        for t in range(nfull):
            pltpu.async_copy(rows0, acc_sh.at[pl.ds(zbase + t * C, C)], zs)
        pltpu.async_copy(rows0.at[pl.ds(0, rem)],
                         acc_sh.at[pl.ds(zbase + nfull * C, rem)], zs)
        for t in range(nfull):
            pltpu.make_async_copy(
                rows0, acc_sh.at[pl.ds(zbase + t * C, C)], zs).wait()
        pltpu.make_async_copy(rows0.at[pl.ds(0, rem)],
                              acc_sh.at[pl.ds(zbase + nfull * C, rem)],
                              zs).wait()

    with jax.named_scope("zero_barrier"):
        plsc.subcore_barrier()

    HF = C // 2

    def _launch(j, b):
        pltpu.async_copy(x_hbm.at[src_v.at[pl.ds(j * C, HF)]],
                         rows[b].at[pl.ds(0, HF)], gsem[b])
        pltpu.async_copy(x_hbm.at[src_v.at[pl.ds(j * C + HF, HF)]],
                         rows[b].at[pl.ds(HF, HF)], gsem[b])
        pltpu.async_copy(dst_hbm.at[pl.ds(base + j * C, HF)], dsta[b],
                         dsem[b])
        pltpu.async_copy(dst_hbm.at[pl.ds(base + j * C + HF, HF)], dstb[b],
                         dsem[b])
        pltpu.async_copy(val_hbm.at[pl.ds(base + j * C, C)], val[b], vsem[b])

    def _scale(b):
        def _group(g, c2):
            vv = val[b][pl.ds(g * L, L)]
            for i in range(L):
                e = g * L + i
                v = vv[i]
                for k in range(D // L):
                    sl = pl.ds(k * L, L)
                    rows[b][e, sl] = rows[b][e, sl] * v
            return c2

        lax.fori_loop(0, C // L, _group, 0)

    def _process(b, j):
        for h in range(2):
            pltpu.make_async_copy(x_hbm.at[src_v.at[pl.ds(j * C + h * HF,
                                                          HF)]],
                                  rows[b].at[pl.ds(h * HF, HF)],
                                  gsem[b]).wait()
        pltpu.make_async_copy(dst_hbm.at[pl.ds(base + j * C, HF)], dsta[b],
                              dsem[b]).wait()
        pltpu.make_async_copy(dst_hbm.at[pl.ds(base + j * C + HF, HF)],
                              dstb[b], dsem[b]).wait()
        pltpu.make_async_copy(val_hbm.at[pl.ds(base + j * C, C)], val[b],
                              vsem[b]).wait()
        _scale(b)
        pltpu.async_copy(rows[b].at[pl.ds(0, HF)], acc_sh.at[dsta[b]],
                         ssem[b], add=True)
        pltpu.async_copy(rows[b].at[pl.ds(HF, HF)], acc_sh.at[dstb[b]],
                         ssem[b], add=True)

    def _drain(b):
        pltpu.make_async_copy(rows[b].at[pl.ds(0, HF)], acc_sh.at[dsta[b]],
                              ssem[b]).wait()
        pltpu.make_async_copy(rows[b].at[pl.ds(HF, HF)], acc_sh.at[dstb[b]],
                              ssem[b]).wait()

    _launch(0, 0)
    _launch(1, 1)

    def _pair(m, carry):
        j0 = 2 * m
        _process(0, j0)
        _process(1, j0 + 1)
        _drain(0)

        @pl.when(j0 + 2 < NCHUNK)
        def _next0():
            _launch(j0 + 2, 0)

        _drain(1)

        @pl.when(j0 + 3 < NCHUNK)
        def _next1():
            _launch(j0 + 3, 1)

        return carry

    with jax.named_scope("edge_loop"):
        lax.fori_loop(0, NPAIR, _pair, 0)

    with jax.named_scope("tail"):
        pltpu.sync_copy(dst_hbm.at[pl.ds(base + TAIL, TAILN)], dstt)
        pltpu.sync_copy(val_hbm.at[pl.ds(base + TAIL, TAILN)], valt)
        pltpu.async_copy(x_hbm.at[src_v.at[pl.ds(TAIL, TAILN)]],
                         rows0.at[pl.ds(0, TAILN)], g0).wait()
        vv = valt[...]
        for i in range(TAILN):
            v = vv[i]
            for k in range(D // L):
                sl = pl.ds(k * L, L)
                rows0[i, sl] = rows0[i, sl] * v
        pltpu.sync_copy(rows0.at[pl.ds(0, TAILN)], acc_sh.at[dstt], add=True)

    with jax.named_scope("end_barrier"):
        plsc.subcore_barrier()

    @pl.when(sid < NS - 1)
    def _pub():
        with jax.named_scope("publish"):
            pltpu.sync_copy(acc_sh.at[pl.ds(sid * SLAB, SLAB)],
                            out_hbm.at[cid, pl.ds(sid * SLAB, SLAB)])

    @pl.when(sid == NS - 1)
    def _pub_last():
        pltpu.sync_copy(acc_sh.at[pl.ds((NS - 1) * SLAB, SLAB_LAST)],
                        out_hbm.at[cid, pl.ds((NS - 1) * SLAB, SLAB_LAST)])


_BR = 2000


def _inter_body(p_ref, f_ref, o_ref):
    o_ref[...] = (p_ref[0] + p_ref[1]) * f_ref[...]


_inter = pl.pallas_call(
    _inter_body,
    grid=(N // _BR,),
    in_specs=[
        pl.BlockSpec((NC, _BR, D), lambda i: (0, i, 0)),
        pl.BlockSpec((_BR, D), lambda i: (i, 0)),
    ],
    out_specs=pl.BlockSpec((_BR, D), lambda i: (i, 0)),
    out_shape=jax.ShapeDtypeStruct((N, D), jnp.float32),
)


def _part1_body(p_ref, f_ref, w1_ref, b_ref, o_ref):
    x1 = p_ref[0] + p_ref[1] + f_ref[...]
    y = lax.dot_general(x1, w1_ref[...], (((1,), (1,)), ((), ())),
                        preferred_element_type=jnp.float32)
    o_ref[...] = y + b_ref[...]


_part1 = pl.pallas_call(
    _part1_body,
    grid=(N // _BR,),
    in_specs=[
        pl.BlockSpec((NC, _BR, D), lambda i: (0, i, 0)),
        pl.BlockSpec((_BR, D), lambda i: (i, 0)),
        pl.BlockSpec((D, D), lambda i: (0, 0)),
        pl.BlockSpec((1, D), lambda i: (0, 0)),
    ],
    out_specs=pl.BlockSpec((_BR, D), lambda i: (i, 0)),
    out_shape=jax.ShapeDtypeStruct((N, D), jnp.float32),
)


def _part2_body(a_ref, q_ref, w2_ref, o_ref):
    x2 = q_ref[0] + q_ref[1]
    y = lax.dot_general(x2, w2_ref[...], (((1,), (1,)), ((), ())),
                        preferred_element_type=jnp.float32)
    o_ref[...] = a_ref[...] + y


_part2 = pl.pallas_call(
    _part2_body,
    grid=(N // _BR,),
    in_specs=[
        pl.BlockSpec((_BR, D), lambda i: (i, 0)),
        pl.BlockSpec((NC, _BR, D), lambda i: (0, i, 0)),
        pl.BlockSpec((D, D), lambda i: (0, 0)),
    ],
    out_specs=pl.BlockSpec((_BR, D), lambda i: (i, 0)),
    out_shape=jax.ShapeDtypeStruct((N, D), jnp.float32),
)


def kernel(laplacian_indices, laplacian_values, features, W1, b1, W2, b2):
    dst_arr = laplacian_indices[0]
    src_arr = laplacian_indices[1]
    p = _spmm_sc(dst_arr, src_arr, laplacian_values, features)
    inter = _inter(p, features)
    parta = _part1(p, features, W1, (b1 + b2).reshape(1, D))
    q = _spmm_sc(dst_arr, src_arr, laplacian_values, inter)
    return _part2(parta, q, W2)


# speedup vs baseline: 10.7898x; 1.0292x over previous
import functools

import jax
import jax.numpy as jnp
from jax import lax
from jax.experimental import pallas as pl
from jax.experimental.pallas import tpu as pltpu
from jax.experimental.pallas import tpu_sc as plsc

N = 10000
E = 320000
D = 128
L = 16
NC, NS = 2, 16
NW = NC * NS
EPW = E // NW
C = 128
NCHUNK = EPW // C
NPAIR = NCHUNK // 2
TAIL = NCHUNK * C
TAILN = EPW - TAIL
SLAB = 640
SLAB_LAST = N - (NS - 1) * SLAB

_mesh = plsc.VectorSubcoreMesh(
    core_axis_name="c", subcore_axis_name="s", num_cores=NC, num_subcores=NS
)


@functools.partial(
    pl.kernel,
    out_type=jax.ShapeDtypeStruct((NC, N, D), jnp.float32),
    mesh=_mesh,
    scratch_types=[
        pltpu.VMEM((EPW,), jnp.int32),
        pltpu.VMEM((C, D), jnp.float32),
        pltpu.VMEM((C, D), jnp.float32),
        pltpu.VMEM((C // 2,), jnp.int32),
        pltpu.VMEM((C // 2,), jnp.int32),
        pltpu.VMEM((C // 2,), jnp.int32),
        pltpu.VMEM((C // 2,), jnp.int32),
        pltpu.VMEM((C,), jnp.float32),
        pltpu.VMEM((C,), jnp.float32),
        pltpu.VMEM((TAILN,), jnp.int32),
        pltpu.VMEM((TAILN,), jnp.float32),
        pltpu.VMEM_SHARED((N, D), jnp.float32),
        pltpu.SemaphoreType.DMA,
        pltpu.SemaphoreType.DMA,
        pltpu.SemaphoreType.DMA,
        pltpu.SemaphoreType.DMA,
        pltpu.SemaphoreType.DMA,
        pltpu.SemaphoreType.DMA,
        pltpu.SemaphoreType.DMA,
        pltpu.SemaphoreType.DMA,
        pltpu.SemaphoreType.DMA,
    ],
)
def _spmm_sc(idx_hbm, val_hbm, x_hbm, out_hbm,
             src_v, rows0, rows1, da0, db0, da1, db1, val0, val1, dstt, valt,
             acc_sh, g0, g1, s0, s1, d0, d1, v0, v1, zs):
    cid = lax.axis_index("c")
    sid = lax.axis_index("s")
    wid = sid * NC + cid
    base = wid * EPW

    rows = (rows0, rows1)
    dsta = (da0, da1)
    dstb = (db0, db1)
    val = (val0, val1)
    gsem = (g0, g1)
    ssem = (s0, s1)
    dsem = (d0, d1)
    vsem = (v0, v1)

    with jax.named_scope("stage_src"):
        pltpu.sync_copy(idx_hbm.at[pl.ds(E + base, EPW)], src_v)

    zvec = jnp.zeros((L,), jnp.float32)

    def _zrow(i, carry):
        for k in range(D // L):
            rows0[i, pl.ds(k * L, L)] = zvec
        return carry

    lax.fori_loop(0, C, _zrow, 0)

    @pl.when(sid < NS - 1)
    def _zero_main():
        for t in range(SLAB // C):
            pltpu.async_copy(
                rows0, acc_sh.at[pl.ds(sid * SLAB + t * C, C)], zs)
        for t in range(SLAB // C):
            pltpu.make_async_copy(
                rows0, acc_sh.at[pl.ds(sid * SLAB + t * C, C)], zs).wait()

    @pl.when(sid == NS - 1)
    def _zero_last():
        zbase = (NS - 1) * SLAB
        nfull = SLAB_LAST // C
        rem = SLAB_LAST % C
        for t in range(nfull):
            pltpu.async_copy(rows0, acc_sh.at[pl.ds(zbase + t * C, C)], zs)
        pltpu.async_copy(rows0.at[pl.ds(0, rem)],
                         acc_sh.at[pl.ds(zbase + nfull * C, rem)], zs)
        for t in range(nfull):
            pltpu.make_async_copy(
                rows0, acc_sh.at[pl.ds(zbase + t * C, C)], zs).wait()
        pltpu.make_async_copy(rows0.at[pl.ds(0, rem)],
                              acc_sh.at[pl.ds(zbase + nfull * C, rem)],
                              zs).wait()

    with jax.named_scope("zero_barrier"):
        plsc.subcore_barrier()

    HF = C // 2

    def _launch(j, b):
        pltpu.async_copy(x_hbm.at[src_v.at[pl.ds(j * C, HF)]],
                         rows[b].at[pl.ds(0, HF)], gsem[b])
        pltpu.async_copy(x_hbm.at[src_v.at[pl.ds(j * C + HF, HF)]],
                         rows[b].at[pl.ds(HF, HF)], gsem[b])
        pltpu.async_copy(idx_hbm.at[pl.ds(base + j * C, HF)], dsta[b],
                         dsem[b])
        pltpu.async_copy(idx_hbm.at[pl.ds(base + j * C + HF, HF)], dstb[b],
                         dsem[b])
        pltpu.async_copy(val_hbm.at[pl.ds(base + j * C, C)], val[b], vsem[b])

    def _scale(b):
        def _group(g, c2):
            vv = val[b][pl.ds(g * L, L)]
            for i in range(L):
                e = g * L + i
                v = vv[i]
                for k in range(D // L):
                    sl = pl.ds(k * L, L)
                    rows[b][e, sl] = rows[b][e, sl] * v
            return c2

        lax.fori_loop(0, C // L, _group, 0)

    def _process(b, j):
        for h in range(2):
            pltpu.make_async_copy(x_hbm.at[src_v.at[pl.ds(j * C + h * HF,
                                                          HF)]],
                                  rows[b].at[pl.ds(h * HF, HF)],
                                  gsem[b]).wait()
        pltpu.make_async_copy(idx_hbm.at[pl.ds(base + j * C, HF)], dsta[b],
                              dsem[b]).wait()
        pltpu.make_async_copy(idx_hbm.at[pl.ds(base + j * C + HF, HF)],
                              dstb[b], dsem[b]).wait()
        pltpu.make_async_copy(val_hbm.at[pl.ds(base + j * C, C)], val[b],
                              vsem[b]).wait()
        _scale(b)
        pltpu.async_copy(rows[b].at[pl.ds(0, HF)], acc_sh.at[dsta[b]],
                         ssem[b], add=True)
        pltpu.async_copy(rows[b].at[pl.ds(HF, HF)], acc_sh.at[dstb[b]],
                         ssem[b], add=True)

    def _drain(b):
        pltpu.make_async_copy(rows[b].at[pl.ds(0, HF)], acc_sh.at[dsta[b]],
                              ssem[b]).wait()
        pltpu.make_async_copy(rows[b].at[pl.ds(HF, HF)], acc_sh.at[dstb[b]],
                              ssem[b]).wait()

    _launch(0, 0)
    _launch(1, 1)

    def _pair(m, carry):
        j0 = 2 * m
        _process(0, j0)
        _process(1, j0 + 1)
        _drain(0)

        @pl.when(j0 + 2 < NCHUNK)
        def _next0():
            _launch(j0 + 2, 0)

        _drain(1)

        @pl.when(j0 + 3 < NCHUNK)
        def _next1():
            _launch(j0 + 3, 1)

        return carry

    with jax.named_scope("edge_loop"):
        lax.fori_loop(0, NPAIR, _pair, 0)

    with jax.named_scope("tail"):
        pltpu.sync_copy(idx_hbm.at[pl.ds(base + TAIL, TAILN)], dstt)
        pltpu.sync_copy(val_hbm.at[pl.ds(base + TAIL, TAILN)], valt)
        pltpu.async_copy(x_hbm.at[src_v.at[pl.ds(TAIL, TAILN)]],
                         rows0.at[pl.ds(0, TAILN)], g0).wait()
        vv = valt[...]
        for i in range(TAILN):
            v = vv[i]
            for k in range(D // L):
                sl = pl.ds(k * L, L)
                rows0[i, sl] = rows0[i, sl] * v
        pltpu.sync_copy(rows0.at[pl.ds(0, TAILN)], acc_sh.at[dstt], add=True)

    with jax.named_scope("end_barrier"):
        plsc.subcore_barrier()

    @pl.when(sid < NS - 1)
    def _pub():
        with jax.named_scope("publish"):
            pltpu.sync_copy(acc_sh.at[pl.ds(sid * SLAB, SLAB)],
                            out_hbm.at[cid, pl.ds(sid * SLAB, SLAB)])

    @pl.when(sid == NS - 1)
    def _pub_last():
        pltpu.sync_copy(acc_sh.at[pl.ds((NS - 1) * SLAB, SLAB_LAST)],
                        out_hbm.at[cid, pl.ds((NS - 1) * SLAB, SLAB_LAST)])


_BR = 2000


def _inter_body(p_ref, f_ref, o_ref):
    o_ref[...] = (p_ref[0] + p_ref[1]) * f_ref[...]


_inter = pl.pallas_call(
    _inter_body,
    grid=(N // _BR,),
    in_specs=[
        pl.BlockSpec((NC, _BR, D), lambda i: (0, i, 0)),
        pl.BlockSpec((_BR, D), lambda i: (i, 0)),
    ],
    out_specs=pl.BlockSpec((_BR, D), lambda i: (i, 0)),
    out_shape=jax.ShapeDtypeStruct((N, D), jnp.float32),
)


def _part1_body(p_ref, f_ref, w1_ref, b_ref, o_ref):
    x1 = p_ref[0] + p_ref[1] + f_ref[...]
    y = lax.dot_general(x1, w1_ref[...], (((1,), (1,)), ((), ())),
                        preferred_element_type=jnp.float32)
    o_ref[...] = y + b_ref[...]


_part1 = pl.pallas_call(
    _part1_body,
    grid=(N // _BR,),
    in_specs=[
        pl.BlockSpec((NC, _BR, D), lambda i: (0, i, 0)),
        pl.BlockSpec((_BR, D), lambda i: (i, 0)),
        pl.BlockSpec((D, D), lambda i: (0, 0)),
        pl.BlockSpec((1, D), lambda i: (0, 0)),
    ],
    out_specs=pl.BlockSpec((_BR, D), lambda i: (i, 0)),
    out_shape=jax.ShapeDtypeStruct((N, D), jnp.float32),
)


def _part2_body(a_ref, q_ref, w2_ref, o_ref):
    x2 = q_ref[0] + q_ref[1]
    y = lax.dot_general(x2, w2_ref[...], (((1,), (1,)), ((), ())),
                        preferred_element_type=jnp.float32)
    o_ref[...] = a_ref[...] + y


_part2 = pl.pallas_call(
    _part2_body,
    grid=(N // _BR,),
    in_specs=[
        pl.BlockSpec((_BR, D), lambda i: (i, 0)),
        pl.BlockSpec((NC, _BR, D), lambda i: (0, i, 0)),
        pl.BlockSpec((D, D), lambda i: (0, 0)),
    ],
    out_specs=pl.BlockSpec((_BR, D), lambda i: (i, 0)),
    out_shape=jax.ShapeDtypeStruct((N, D), jnp.float32),
)


def kernel(laplacian_indices, laplacian_values, features, W1, b1, W2, b2):
    idx_flat = laplacian_indices.reshape(2 * E)
    p = _spmm_sc(idx_flat, laplacian_values, features)
    inter = _inter(p, features)
    parta = _part1(p, features, W1, (b1 + b2).reshape(1, D))
    q = _spmm_sc(idx_flat, laplacian_values, inter)
    return _part2(parta, q, W2)


# speedup vs baseline: 10.8641x; 1.0069x over previous
import functools

import jax
import jax.numpy as jnp
from jax import lax
from jax.experimental import pallas as pl
from jax.experimental.pallas import tpu as pltpu
from jax.experimental.pallas import tpu_sc as plsc

N = 10000
E = 320000
D = 128
L = 16
NC, NS = 2, 16
NW = NC * NS
EPW = E // NW
C = 128
NCHUNK = EPW // C
NPAIR = NCHUNK // 2
TAIL = NCHUNK * C
TAILN = EPW - TAIL
SLAB = 640
SLAB_LAST = N - (NS - 1) * SLAB

_mesh = plsc.VectorSubcoreMesh(
    core_axis_name="c", subcore_axis_name="s", num_cores=NC, num_subcores=NS
)


@functools.partial(
    pl.kernel,
    out_type=jax.ShapeDtypeStruct((NC, N, D), jnp.float32),
    mesh=_mesh,
    scratch_types=[
        pltpu.VMEM((EPW,), jnp.int32),
        pltpu.VMEM((C, D), jnp.float32),
        pltpu.VMEM((C, D), jnp.float32),
        pltpu.VMEM((C // 2,), jnp.int32),
        pltpu.VMEM((C // 2,), jnp.int32),
        pltpu.VMEM((C // 2,), jnp.int32),
        pltpu.VMEM((C // 2,), jnp.int32),
        pltpu.VMEM((C,), jnp.float32),
        pltpu.VMEM((C,), jnp.float32),
        pltpu.VMEM((TAILN,), jnp.int32),
        pltpu.VMEM((TAILN,), jnp.float32),
        pltpu.VMEM_SHARED((N, D), jnp.float32),
        pltpu.SemaphoreType.DMA,
        pltpu.SemaphoreType.DMA,
        pltpu.SemaphoreType.DMA,
        pltpu.SemaphoreType.DMA,
        pltpu.SemaphoreType.DMA,
        pltpu.SemaphoreType.DMA,
        pltpu.SemaphoreType.DMA,
        pltpu.SemaphoreType.DMA,
        pltpu.SemaphoreType.DMA,
        pltpu.SemaphoreType.DMA,
    ],
)
def _spmm_sc(idx_hbm, val_hbm, x_hbm, out_hbm,
             src_v, rows0, rows1, da0, db0, da1, db1, val0, val1, dstt, valt,
             acc_sh, g0, g1, s0, s1, d0, d1, v0, v1, zs, ss):
    cid = lax.axis_index("c")
    sid = lax.axis_index("s")
    wid = sid * NC + cid
    base = wid * EPW

    rows = (rows0, rows1)
    dsta = (da0, da1)
    dstb = (db0, db1)
    val = (val0, val1)
    gsem = (g0, g1)
    ssem = (s0, s1)
    dsem = (d0, d1)
    vsem = (v0, v1)

    pltpu.async_copy(idx_hbm.at[pl.ds(E + base, EPW)], src_v, ss)

    zvec = jnp.zeros((L,), jnp.float32)

    def _zrow(i, carry):
        for k in range(D // L):
            rows0[i, pl.ds(k * L, L)] = zvec
        return carry

    lax.fori_loop(0, C, _zrow, 0)

    @pl.when(sid < NS - 1)
    def _zero_main():
        for t in range(SLAB // C):
            pltpu.async_copy(
                rows0, acc_sh.at[pl.ds(sid * SLAB + t * C, C)], zs)
        for t in range(SLAB // C):
            pltpu.make_async_copy(
                rows0, acc_sh.at[pl.ds(sid * SLAB + t * C, C)], zs).wait()

    @pl.when(sid == NS - 1)
    def _zero_last():
        zbase = (NS - 1) * SLAB
        nfull = SLAB_LAST // C
        rem = SLAB_LAST % C
        for t in range(nfull):
            pltpu.async_copy(rows0, acc_sh.at[pl.ds(zbase + t * C, C)], zs)
        pltpu.async_copy(rows0.at[pl.ds(0, rem)],
                         acc_sh.at[pl.ds(zbase + nfull * C, rem)], zs)
        for t in range(nfull):
            pltpu.make_async_copy(
                rows0, acc_sh.at[pl.ds(zbase + t * C, C)], zs).wait()
        pltpu.make_async_copy(rows0.at[pl.ds(0, rem)],
                              acc_sh.at[pl.ds(zbase + nfull * C, rem)],
                              zs).wait()

    with jax.named_scope("stage_src"):
        pltpu.make_async_copy(idx_hbm.at[pl.ds(E + base, EPW)], src_v,
                              ss).wait()

    HF = C // 2

    def _launch(j, b):
        pltpu.async_copy(x_hbm.at[src_v.at[pl.ds(j * C, HF)]],
                         rows[b].at[pl.ds(0, HF)], gsem[b])
        pltpu.async_copy(x_hbm.at[src_v.at[pl.ds(j * C + HF, HF)]],
                         rows[b].at[pl.ds(HF, HF)], gsem[b])
        pltpu.async_copy(idx_hbm.at[pl.ds(base + j * C, HF)], dsta[b],
                         dsem[b])
        pltpu.async_copy(idx_hbm.at[pl.ds(base + j * C + HF, HF)], dstb[b],
                         dsem[b])
        pltpu.async_copy(val_hbm.at[pl.ds(base + j * C, C)], val[b], vsem[b])

    def _scale(b):
        def _group(g, c2):
            vv = val[b][pl.ds(g * L, L)]
            for i in range(L):
                e = g * L + i
                v = vv[i]
                for k in range(D // L):
                    sl = pl.ds(k * L, L)
                    rows[b][e, sl] = rows[b][e, sl] * v
            return c2

        lax.fori_loop(0, C // L, _group, 0)

    def _process(b, j):
        for h in range(2):
            pltpu.make_async_copy(x_hbm.at[src_v.at[pl.ds(j * C + h * HF,
                                                          HF)]],
                                  rows[b].at[pl.ds(h * HF, HF)],
                                  gsem[b]).wait()
        pltpu.make_async_copy(idx_hbm.at[pl.ds(base + j * C, HF)], dsta[b],
                              dsem[b]).wait()
        pltpu.make_async_copy(idx_hbm.at[pl.ds(base + j * C + HF, HF)],
                              dstb[b], dsem[b]).wait()
        pltpu.make_async_copy(val_hbm.at[pl.ds(base + j * C, C)], val[b],
                              vsem[b]).wait()
        _scale(b)
        pltpu.async_copy(rows[b].at[pl.ds(0, HF)], acc_sh.at[dsta[b]],
                         ssem[b], add=True)
        pltpu.async_copy(rows[b].at[pl.ds(HF, HF)], acc_sh.at[dstb[b]],
                         ssem[b], add=True)

    def _drain(b):
        pltpu.make_async_copy(rows[b].at[pl.ds(0, HF)], acc_sh.at[dsta[b]],
                              ssem[b]).wait()
        pltpu.make_async_copy(rows[b].at[pl.ds(HF, HF)], acc_sh.at[dstb[b]],
                              ssem[b]).wait()

    _launch(0, 0)
    _launch(1, 1)
    with jax.named_scope("zero_barrier"):
        plsc.subcore_barrier()

    def _pair(m, carry):
        j0 = 2 * m
        _process(0, j0)
        _process(1, j0 + 1)
        _drain(0)

        @pl.when(j0 + 2 < NCHUNK)
        def _next0():
            _launch(j0 + 2, 0)

        _drain(1)

        @pl.when(j0 + 3 < NCHUNK)
        def _next1():
            _launch(j0 + 3, 1)

        return carry

    with jax.named_scope("edge_loop"):
        lax.fori_loop(0, NPAIR, _pair, 0)

    with jax.named_scope("tail"):
        pltpu.sync_copy(idx_hbm.at[pl.ds(base + TAIL, TAILN)], dstt)
        pltpu.sync_copy(val_hbm.at[pl.ds(base + TAIL, TAILN)], valt)
        pltpu.async_copy(x_hbm.at[src_v.at[pl.ds(TAIL, TAILN)]],
                         rows0.at[pl.ds(0, TAILN)], g0).wait()
        vv = valt[...]
        for i in range(TAILN):
            v = vv[i]
            for k in range(D // L):
                sl = pl.ds(k * L, L)
                rows0[i, sl] = rows0[i, sl] * v
        pltpu.sync_copy(rows0.at[pl.ds(0, TAILN)], acc_sh.at[dstt], add=True)

    with jax.named_scope("end_barrier"):
        plsc.subcore_barrier()

    @pl.when(sid < NS - 1)
    def _pub():
        with jax.named_scope("publish"):
            pltpu.sync_copy(acc_sh.at[pl.ds(sid * SLAB, SLAB)],
                            out_hbm.at[cid, pl.ds(sid * SLAB, SLAB)])

    @pl.when(sid == NS - 1)
    def _pub_last():
        pltpu.sync_copy(acc_sh.at[pl.ds((NS - 1) * SLAB, SLAB_LAST)],
                        out_hbm.at[cid, pl.ds((NS - 1) * SLAB, SLAB_LAST)])


_BR = 2000


def _inter_body(p_ref, f_ref, o_ref):
    o_ref[...] = (p_ref[0] + p_ref[1]) * f_ref[...]


_inter = pl.pallas_call(
    _inter_body,
    grid=(N // _BR,),
    in_specs=[
        pl.BlockSpec((NC, _BR, D), lambda i: (0, i, 0)),
        pl.BlockSpec((_BR, D), lambda i: (i, 0)),
    ],
    out_specs=pl.BlockSpec((_BR, D), lambda i: (i, 0)),
    out_shape=jax.ShapeDtypeStruct((N, D), jnp.float32),
)


def _part1_body(p_ref, f_ref, w1_ref, b_ref, o_ref):
    x1 = p_ref[0] + p_ref[1] + f_ref[...]
    y = lax.dot_general(x1, w1_ref[...], (((1,), (1,)), ((), ())),
                        preferred_element_type=jnp.float32)
    o_ref[...] = y + b_ref[...]


_part1 = pl.pallas_call(
    _part1_body,
    grid=(N // _BR,),
    in_specs=[
        pl.BlockSpec((NC, _BR, D), lambda i: (0, i, 0)),
        pl.BlockSpec((_BR, D), lambda i: (i, 0)),
        pl.BlockSpec((D, D), lambda i: (0, 0)),
        pl.BlockSpec((1, D), lambda i: (0, 0)),
    ],
    out_specs=pl.BlockSpec((_BR, D), lambda i: (i, 0)),
    out_shape=jax.ShapeDtypeStruct((N, D), jnp.float32),
)


def _part2_body(a_ref, q_ref, w2_ref, o_ref):
    x2 = q_ref[0] + q_ref[1]
    y = lax.dot_general(x2, w2_ref[...], (((1,), (1,)), ((), ())),
                        preferred_element_type=jnp.float32)
    o_ref[...] = a_ref[...] + y


_part2 = pl.pallas_call(
    _part2_body,
    grid=(N // _BR,),
    in_specs=[
        pl.BlockSpec((_BR, D), lambda i: (i, 0)),
        pl.BlockSpec((NC, _BR, D), lambda i: (0, i, 0)),
        pl.BlockSpec((D, D), lambda i: (0, 0)),
    ],
    out_specs=pl.BlockSpec((_BR, D), lambda i: (i, 0)),
    out_shape=jax.ShapeDtypeStruct((N, D), jnp.float32),
)


def kernel(laplacian_indices, laplacian_values, features, W1, b1, W2, b2):
    idx_flat = laplacian_indices.reshape(2 * E)
    p = _spmm_sc(idx_flat, laplacian_values, features)
    inter = _inter(p, features)
    parta = _part1(p, features, W1, (b1 + b2).reshape(1, D))
    q = _spmm_sc(idx_flat, laplacian_values, inter)
    return _part2(parta, q, W2)
